# Initial kernel scaffold; baseline (speedup 1.0000x reference)
import jax
import jax.numpy as jnp
from jax.experimental import pallas as pl


def kernel(x, edge_bf, edge_h, edge_index, W1, b1, W2, b2, W3, b3, W4, b4):
    raise NotImplementedError("write your pallas kernel here")


# speedup vs baseline: 1.8349x; 1.8349x over previous
import numpy as np
import jax
import jax.numpy as jnp
from jax import lax
from jax.experimental import pallas as pl
from jax.experimental.pallas import tpu as pltpu
from jax.experimental.pallas import tpu_sc as plsc

_N_NODES = 50000
_N_EDGES = 800000
_IN = 64
_RAD = 128
_HALF = 32
_LOG2 = float(np.log(2.0))

_E_PAD = 802816
_IDX_ROWS = _E_PAD // 128
_ROWS_PER_SUB = _IDX_ROWS // 16
_KR = 2
_CHUNK = _KR * 128
_CHUNKS = _ROWS_PER_SUB // _KR
_N_PAD = 50048
_DUMP = _N_NODES

_EBLK = 2000
_NBLK = 5000


def _ssp(v):
    return jnp.maximum(v, 0.0) + jnp.log1p(jnp.exp(-jnp.abs(v))) - _LOG2


def _edge_mlp_body(bf_ref, eh_ref, w1_ref, b1_ref, w2_ref, b2_ref,
                   o0_ref, o1_ref):
    t = jnp.dot(bf_ref[...], w1_ref[...], preferred_element_type=jnp.float32)
    t = _ssp(t + b1_ref[...])
    t = jnp.dot(t, w2_ref[...], preferred_element_type=jnp.float32)
    t = _ssp(t + b2_ref[...]) * eh_ref[...]
    o0_ref[...] = t[:, :_HALF]
    o1_ref[...] = t[:, _HALF:]


def _edge_mlp(edge_bf, edge_h, W1, b1, W2, b2):
    grid = _N_EDGES // _EBLK
    return pl.pallas_call(
        _edge_mlp_body,
        grid=(grid,),
        in_specs=[
            pl.BlockSpec((_EBLK, _RAD), lambda i: (i, 0)),
            pl.BlockSpec((_EBLK, _IN), lambda i: (i, 0)),
            pl.BlockSpec((_RAD, _IN), lambda i: (0, 0)),
            pl.BlockSpec((1, _IN), lambda i: (0, 0)),
            pl.BlockSpec((_IN, _IN), lambda i: (0, 0)),
            pl.BlockSpec((1, _IN), lambda i: (0, 0)),
        ],
        out_specs=[
            pl.BlockSpec((_EBLK, _HALF), lambda i: (i, 0)),
            pl.BlockSpec((_EBLK, _HALF), lambda i: (i, 0)),
        ],
        out_shape=[
            jax.ShapeDtypeStruct((_E_PAD, _HALF), jnp.float32),
            jax.ShapeDtypeStruct((_E_PAD, _HALF), jnp.float32),
        ],
    )(edge_bf, edge_h, W1, b1.reshape(1, _IN), W2, b2.reshape(1, _IN))


def _sc_body(xs0, xs1, eh0, eh1, src2, dst2, z32, z1, o1,
             s0_out, s1_out, c0_out, c1_out,
             sidx, didx, rows, ehv, ones_v, s_sh, cnt_sh):
    cid = lax.axis_index("c")
    sid = lax.axis_index("s")

    wr = _N_PAD // 16
    pltpu.sync_copy(z32.at[pl.ds(sid * wr, wr)], s_sh.at[pl.ds(sid * wr, wr)])
    pltpu.sync_copy(z1.at[pl.ds(sid * wr, wr)], cnt_sh.at[pl.ds(sid * wr, wr)])
    pltpu.sync_copy(o1, ones_v)
    plsc.subcore_barrier()

    def do_half(x_tab, eh_tab, parity):
        def chunk_body(t, carry):
            rb = sid * _ROWS_PER_SUB + t * _KR
            eb = rb * 128
            pltpu.sync_copy(src2.at[pl.ds(rb, _KR)], sidx)
            pltpu.sync_copy(dst2.at[pl.ds(rb, _KR)], didx)
            pltpu.sync_copy(eh_tab.at[pl.ds(eb, _CHUNK)], ehv)
            for j in range(_KR):
                pltpu.sync_copy(x_tab.at[sidx.at[j]],
                                rows.at[pl.ds(j * 128, 128)])

            def mul_body(i, c):
                rows[i, pl.ds(0, 16)] = rows[i, pl.ds(0, 16)] * ehv[i, pl.ds(0, 16)]
                rows[i, pl.ds(16, 16)] = rows[i, pl.ds(16, 16)] * ehv[i, pl.ds(16, 16)]
                return c

            lax.fori_loop(0, _CHUNK, mul_body, 0)

            for j in range(_KR):
                pltpu.sync_copy(rows.at[pl.ds(j * 128, 128)],
                                s_sh.at[didx.at[j]], add=True)
            for j in range(_KR):
                if j % 2 == parity:
                    pltpu.sync_copy(ones_v, cnt_sh.at[didx.at[j]], add=True)
            return carry

        lax.fori_loop(0, _CHUNKS, chunk_body, 0)

    @pl.when(cid == 0)
    def _():
        do_half(xs0, eh0, 0)

    @pl.when(cid == 1)
    def _():
        do_half(xs1, eh1, 1)

    plsc.subcore_barrier()

    @pl.when(cid == 0)
    def _():
        pltpu.sync_copy(s_sh.at[pl.ds(sid * wr, wr)],
                        s0_out.at[pl.ds(sid * wr, wr)])
        pltpu.sync_copy(cnt_sh.at[pl.ds(sid * wr, wr)],
                        c0_out.at[pl.ds(sid * wr, wr)])

    @pl.when(cid == 1)
    def _():
        pltpu.sync_copy(s_sh.at[pl.ds(sid * wr, wr)],
                        s1_out.at[pl.ds(sid * wr, wr)])
        pltpu.sync_copy(cnt_sh.at[pl.ds(sid * wr, wr)],
                        c1_out.at[pl.ds(sid * wr, wr)])


def _sc_gather_scatter(xs0, xs1, eh0, eh1, src2, dst2, z32, z1, o1):
    f32 = jnp.float32
    mesh = plsc.VectorSubcoreMesh(core_axis_name="c", subcore_axis_name="s")
    kern = pl.kernel(
        _sc_body,
        compiler_params=pltpu.CompilerParams(use_tc_tiling_on_sc=False),
        out_type=[
            jax.ShapeDtypeStruct((_N_PAD, _HALF), f32),
            jax.ShapeDtypeStruct((_N_PAD, _HALF), f32),
            jax.ShapeDtypeStruct((_N_PAD,), f32),
            jax.ShapeDtypeStruct((_N_PAD,), f32),
        ],
        mesh=mesh,
        scratch_types=[
            pltpu.VMEM((_KR, 128), jnp.int32),
            pltpu.VMEM((_KR, 128), jnp.int32),
            pltpu.VMEM((_CHUNK, _HALF), f32),
            pltpu.VMEM((_CHUNK, _HALF), f32),
            pltpu.VMEM((128,), f32),
            pltpu.VMEM_SHARED((_N_PAD, _HALF), f32),
            pltpu.VMEM_SHARED((_N_PAD,), f32),
        ],
    )
    return kern(xs0, xs1, eh0, eh1, src2, dst2, z32, z1, o1)


def _out_mlp_body(s0_ref, s1_ref, c0_ref, c1_ref, w3_ref, b3_ref,
                  w4_ref, b4_ref, o_ref):
    cnt = c0_ref[...] + c1_ref[...]
    scale = 1.0 / jnp.maximum(cnt, 1.0)
    h = jnp.concatenate([s0_ref[...], s1_ref[...]], axis=1) * scale
    t = jnp.dot(h, w3_ref[...], preferred_element_type=jnp.float32)
    t = _ssp(t + b3_ref[...])
    t = jnp.dot(t, w4_ref[...], preferred_element_type=jnp.float32)
    o_ref[...] = _ssp(t + b4_ref[...])


def _out_mlp(s0, s1, c0, c1, W3, b3, W4, b4):
    grid = _N_NODES // _NBLK
    return pl.pallas_call(
        _out_mlp_body,
        grid=(grid,),
        in_specs=[
            pl.BlockSpec((_NBLK, _HALF), lambda i: (i, 0)),
            pl.BlockSpec((_NBLK, _HALF), lambda i: (i, 0)),
            pl.BlockSpec((_NBLK, 1), lambda i: (i, 0)),
            pl.BlockSpec((_NBLK, 1), lambda i: (i, 0)),
            pl.BlockSpec((_IN, _IN), lambda i: (0, 0)),
            pl.BlockSpec((1, _IN), lambda i: (0, 0)),
            pl.BlockSpec((_IN, _IN), lambda i: (0, 0)),
            pl.BlockSpec((1, _IN), lambda i: (0, 0)),
        ],
        out_specs=pl.BlockSpec((_NBLK, _IN), lambda i: (i, 0)),
        out_shape=jax.ShapeDtypeStruct((_N_NODES, _IN), jnp.float32),
    )(s0, s1, c0, c1, W3, b3.reshape(1, _IN), W4, b4.reshape(1, _IN))


def kernel(x, edge_bf, edge_h, edge_index, W1, b1, W2, b2, W3, b3, W4, b4):
    src = edge_index[0]
    dst = edge_index[1]
    pad = _E_PAD - _N_EDGES
    src_p = jnp.concatenate(
        [src, jnp.zeros((pad,), jnp.int32)]).reshape(_IDX_ROWS, 128)
    dst_p = jnp.concatenate(
        [dst, jnp.full((pad,), _DUMP, jnp.int32)]).reshape(_IDX_ROWS, 128)
    xs0 = x[:, :_HALF]
    xs1 = x[:, _HALF:]

    eh0, eh1 = _edge_mlp(edge_bf, edge_h, W1, b1, W2, b2)

    z32 = jnp.zeros((_N_PAD, _HALF), jnp.float32)
    z1 = jnp.zeros((_N_PAD,), jnp.float32)
    o1 = jnp.ones((128,), jnp.float32)
    s0, s1, c0, c1 = _sc_gather_scatter(
        xs0, xs1, eh0, eh1, src_p, dst_p, z32, z1, o1)
    c0 = c0.reshape(_N_PAD, 1)
    c1 = c1.reshape(_N_PAD, 1)

    return _out_mlp(s0, s1, c0, c1, W3, b3, W4, b4)


# speedup vs baseline: 3.4032x; 1.8547x over previous
import numpy as np
import jax
import jax.numpy as jnp
from jax import lax
from jax.experimental import pallas as pl
from jax.experimental.pallas import tpu as pltpu
from jax.experimental.pallas import tpu_sc as plsc

_N_NODES = 50000
_N_EDGES = 800000
_IN = 64
_RAD = 128
_HALF = 32
_LOG2 = float(np.log(2.0))

_E_PAD = 802816
_IDX_ROWS = _E_PAD // 128
_EP4 = _E_PAD // 4
_ROWS_PER_SUB = _IDX_ROWS // 16
_NCH = 14
_NBLK2 = _ROWS_PER_SUB // _NCH // 2
_N_PAD = 50048
_DUMP = _N_NODES

_EBLK = 3584
_EB4 = _EBLK // 4
_NBLK = 5000


def _ssp(v):
    return jnp.maximum(v, 0.0) + jnp.log1p(jnp.exp(-jnp.abs(v))) - _LOG2


def _edge_mlp_body(bf_ref, eh_ref, w1_ref, b1_ref, w2_ref, b2_ref,
                   o0_ref, o1_ref):
    t = jnp.dot(bf_ref[...], w1_ref[...], preferred_element_type=jnp.float32)
    t = _ssp(t + b1_ref[...])
    t = jnp.dot(t, w2_ref[...], preferred_element_type=jnp.float32)
    t = _ssp(t + b2_ref[...]) * eh_ref[...]
    t0 = t[:, :_HALF]
    t1 = t[:, _HALF:]
    o0_ref[...] = jnp.concatenate(
        [t0[k * _EB4:(k + 1) * _EB4] for k in range(4)], axis=1)
    o1_ref[...] = jnp.concatenate(
        [t1[k * _EB4:(k + 1) * _EB4] for k in range(4)], axis=1)


def _edge_mlp(edge_bf, edge_h, W1, b1, W2, b2):
    grid = _E_PAD // _EBLK
    return pl.pallas_call(
        _edge_mlp_body,
        grid=(grid,),
        in_specs=[
            pl.BlockSpec((_EBLK, _RAD), lambda i: (i, 0)),
            pl.BlockSpec((_EBLK, _IN), lambda i: (i, 0)),
            pl.BlockSpec((_RAD, _IN), lambda i: (0, 0)),
            pl.BlockSpec((1, _IN), lambda i: (0, 0)),
            pl.BlockSpec((_IN, _IN), lambda i: (0, 0)),
            pl.BlockSpec((1, _IN), lambda i: (0, 0)),
        ],
        out_specs=[
            pl.BlockSpec((_EB4, 128), lambda i: (i, 0)),
            pl.BlockSpec((_EB4, 128), lambda i: (i, 0)),
        ],
        out_shape=[
            jax.ShapeDtypeStruct((_EP4, 128), jnp.float32),
            jax.ShapeDtypeStruct((_EP4, 128), jnp.float32),
        ],
    )(edge_bf, edge_h, W1, b1.reshape(1, _IN), W2, b2.reshape(1, _IN))


def _sc_body(xs0, xs1, eh0, eh1, src2, dst2, z32, z1, o1,
             s0_out, s1_out, c0_out, c1_out,
             sidx, didx, rows, ehv, ones_v, s_sh, cnt_sh,
             idx_sem, in_sem0, in_sem1, out_sem0, out_sem1):
    cid = lax.axis_index("c")
    sid = lax.axis_index("s")

    wr = _N_PAD // 16
    pltpu.sync_copy(z32.at[pl.ds(sid * wr, wr)], s_sh.at[pl.ds(sid * wr, wr)])
    pltpu.sync_copy(z1.at[pl.ds(sid * wr, wr)], cnt_sh.at[pl.ds(sid * wr, wr)])
    pltpu.sync_copy(o1, ones_v)
    plsc.subcore_barrier()

    row0 = sid * _ROWS_PER_SUB
    in_sems = (in_sem0, in_sem1)
    out_sems = (out_sem0, out_sem1)

    def fetch_idx(blk, b):
        rb = row0 + blk * _NCH
        pltpu.async_copy(src2.at[pl.ds(rb, _NCH)], sidx.at[b], idx_sem)
        pltpu.async_copy(dst2.at[pl.ds(rb, _NCH)], didx.at[b], idx_sem)

    def wait_idx(b):
        pltpu.make_async_copy(src2.at[pl.ds(0, _NCH)], sidx.at[b],
                              idx_sem).wait()
        pltpu.make_async_copy(dst2.at[pl.ds(0, _NCH)], didx.at[b],
                              idx_sem).wait()

    def do_half(x_tab, eh_tab, parity):

        def process(blk, ib):
            def issue_in(i, bb):
                g = pltpu.async_copy(x_tab.at[sidx.at[ib, i]], rows.at[bb],
                                     in_sems[bb])
                e = pltpu.async_copy(
                    eh_tab.at[pl.ds((row0 + blk * _NCH + i) * 32, 32)],
                    ehv.at[bb], in_sems[bb])
                return (g, e)

            def compute(bb):
                def mb(r, c):
                    for k in range(4):
                        e = r * 4 + k
                        rows[bb, e, pl.ds(0, 16)] = (
                            rows[bb, e, pl.ds(0, 16)]
                            * ehv[bb, r, pl.ds(k * 32, 16)])
                        rows[bb, e, pl.ds(16, 16)] = (
                            rows[bb, e, pl.ds(16, 16)]
                            * ehv[bb, r, pl.ds(k * 32 + 16, 16)])
                    return c
                lax.fori_loop(0, 32, mb, 0)

            descs = {}
            sdesc = {}
            descs[0] = issue_in(0, 0)
            for i in range(_NCH):
                b = i % 2
                nb = 1 - b
                if i < _NCH - 1:
                    if i >= 1:
                        sdesc[nb].wait()
                    descs[nb] = issue_in(i + 1, nb)
                for d in descs[b]:
                    d.wait()
                compute(b)
                sdesc[b] = pltpu.async_copy(rows.at[b],
                                            s_sh.at[didx.at[ib, i]],
                                            out_sems[b], add=True)
                if i % 2 == parity:
                    pltpu.sync_copy(ones_v, cnt_sh.at[didx.at[ib, i]],
                                    add=True)
            sdesc[0].wait()
            sdesc[1].wait()

        def blk2_body(t, c):
            wait_idx(0)
            fetch_idx(2 * t + 1, 1)
            process(2 * t, 0)
            wait_idx(1)

            @pl.when(t < _NBLK2 - 1)
            def _():
                fetch_idx(2 * t + 2, 0)

            process(2 * t + 1, 1)
            return c

        fetch_idx(0, 0)
        lax.fori_loop(0, _NBLK2, blk2_body, 0)

    @pl.when(cid == 0)
    def _():
        do_half(xs0, eh0, 0)

    @pl.when(cid == 1)
    def _():
        do_half(xs1, eh1, 1)

    plsc.subcore_barrier()

    @pl.when(cid == 0)
    def _():
        pltpu.sync_copy(s_sh.at[pl.ds(sid * wr, wr)],
                        s0_out.at[pl.ds(sid * wr, wr)])
        pltpu.sync_copy(cnt_sh.at[pl.ds(sid * wr, wr)],
                        c0_out.at[pl.ds(sid * wr, wr)])

    @pl.when(cid == 1)
    def _():
        pltpu.sync_copy(s_sh.at[pl.ds(sid * wr, wr)],
                        s1_out.at[pl.ds(sid * wr, wr)])
        pltpu.sync_copy(cnt_sh.at[pl.ds(sid * wr, wr)],
                        c1_out.at[pl.ds(sid * wr, wr)])


def _sc_gather_scatter(xs0, xs1, eh0, eh1, src2, dst2, z32, z1, o1):
    f32 = jnp.float32
    mesh = plsc.VectorSubcoreMesh(core_axis_name="c", subcore_axis_name="s")
    kern = pl.kernel(
        _sc_body,
        compiler_params=pltpu.CompilerParams(use_tc_tiling_on_sc=False),
        out_type=[
            jax.ShapeDtypeStruct((_N_PAD, _HALF), f32),
            jax.ShapeDtypeStruct((_N_PAD, _HALF), f32),
            jax.ShapeDtypeStruct((_N_PAD,), f32),
            jax.ShapeDtypeStruct((_N_PAD,), f32),
        ],
        mesh=mesh,
        scratch_types=[
            pltpu.VMEM((2, _NCH, 128), jnp.int32),
            pltpu.VMEM((2, _NCH, 128), jnp.int32),
            pltpu.VMEM((2, 128, _HALF), jnp.float32),
            pltpu.VMEM((2, 32, 128), jnp.float32),
            pltpu.VMEM((128,), jnp.float32),
            pltpu.VMEM_SHARED((_N_PAD, _HALF), f32),
            pltpu.VMEM_SHARED((_N_PAD,), f32),
            pltpu.SemaphoreType.DMA,
            pltpu.SemaphoreType.DMA,
            pltpu.SemaphoreType.DMA,
            pltpu.SemaphoreType.DMA,
            pltpu.SemaphoreType.DMA,
        ],
    )
    return kern(xs0, xs1, eh0, eh1, src2, dst2, z32, z1, o1)


def _out_mlp_body(s0_ref, s1_ref, c0_ref, c1_ref, w3_ref, b3_ref,
                  w4_ref, b4_ref, o_ref):
    cnt = c0_ref[...] + c1_ref[...]
    scale = 1.0 / jnp.maximum(cnt, 1.0)
    h = jnp.concatenate([s0_ref[...], s1_ref[...]], axis=1) * scale
    t = jnp.dot(h, w3_ref[...], preferred_element_type=jnp.float32)
    t = _ssp(t + b3_ref[...])
    t = jnp.dot(t, w4_ref[...], preferred_element_type=jnp.float32)
    o_ref[...] = _ssp(t + b4_ref[...])


def _out_mlp(s0, s1, c0, c1, W3, b3, W4, b4):
    grid = _N_NODES // _NBLK
    return pl.pallas_call(
        _out_mlp_body,
        grid=(grid,),
        in_specs=[
            pl.BlockSpec((_NBLK, _HALF), lambda i: (i, 0)),
            pl.BlockSpec((_NBLK, _HALF), lambda i: (i, 0)),
            pl.BlockSpec((_NBLK, 1), lambda i: (i, 0)),
            pl.BlockSpec((_NBLK, 1), lambda i: (i, 0)),
            pl.BlockSpec((_IN, _IN), lambda i: (0, 0)),
            pl.BlockSpec((1, _IN), lambda i: (0, 0)),
            pl.BlockSpec((_IN, _IN), lambda i: (0, 0)),
            pl.BlockSpec((1, _IN), lambda i: (0, 0)),
        ],
        out_specs=pl.BlockSpec((_NBLK, _IN), lambda i: (i, 0)),
        out_shape=jax.ShapeDtypeStruct((_N_NODES, _IN), jnp.float32),
    )(s0, s1, c0, c1, W3, b3.reshape(1, _IN), W4, b4.reshape(1, _IN))


def kernel(x, edge_bf, edge_h, edge_index, W1, b1, W2, b2, W3, b3, W4, b4):
    src = edge_index[0]
    dst = edge_index[1]
    pad = _E_PAD - _N_EDGES
    def _perm(a):
        a = a.reshape(_E_PAD // _EBLK, 4, _EB4 // 32, 32)
        return a.transpose(0, 2, 3, 1).reshape(_IDX_ROWS, 128)

    src_p = _perm(jnp.concatenate([src, jnp.zeros((pad,), jnp.int32)]))
    dst_p = _perm(jnp.concatenate([dst, jnp.full((pad,), _DUMP, jnp.int32)]))
    xs0 = x[:, :_HALF]
    xs1 = x[:, _HALF:]

    eh0, eh1 = _edge_mlp(edge_bf, edge_h, W1, b1, W2, b2)

    z32 = jnp.zeros((_N_PAD, _HALF), jnp.float32)
    z1 = jnp.zeros((_N_PAD,), jnp.float32)
    o1 = jnp.ones((128,), jnp.float32)
    s0, s1, c0, c1 = _sc_gather_scatter(
        xs0, xs1, eh0, eh1, src_p, dst_p, z32, z1, o1)
    c0 = c0.reshape(_N_PAD, 1)
    c1 = c1.reshape(_N_PAD, 1)

    return _out_mlp(s0, s1, c0, c1, W3, b3, W4, b4)


# speedup vs baseline: 4.4533x; 1.3086x over previous
import numpy as np
import jax
import jax.numpy as jnp
from jax import lax
from jax.experimental import pallas as pl
from jax.experimental.pallas import tpu as pltpu
from jax.experimental.pallas import tpu_sc as plsc

_N_NODES = 50000
_N_EDGES = 800000
_IN = 64
_RAD = 128
_HALF = 32
_LOG2 = float(np.log(2.0))

_E_PAD = 802816
_IDX_ROWS = _E_PAD // 128
_EP4 = _E_PAD // 4
_ROWS_PER_SUB = _IDX_ROWS // 16
_NCH = 14
_NBLK2 = _ROWS_PER_SUB // _NCH // 2
_N_PAD = 50048
_DUMP = _N_NODES

_EBLK = 3584
_EB4 = _EBLK // 4
_NBLK = 5000


def _ssp(v):
    return jnp.maximum(v, 0.0) + jnp.log(1.0 + jnp.exp(-jnp.abs(v))) - _LOG2


def _edge_mlp_body(bf_ref, eht_ref, w1_ref, b1_ref, w2_ref, b2_ref,
                   o0_ref, o1_ref):
    t = jnp.dot(bf_ref[...], w1_ref[...], preferred_element_type=jnp.float32)
    t = _ssp(t + b1_ref[...])
    t = jnp.dot(t, w2_ref[...], preferred_element_type=jnp.float32)
    t = _ssp(t + b2_ref[...]) * eht_ref[...].T
    t0 = t[:, :_HALF]
    t1 = t[:, _HALF:]
    o0_ref[...] = jnp.concatenate(
        [t0[k * _EB4:(k + 1) * _EB4] for k in range(4)], axis=1)
    o1_ref[...] = jnp.concatenate(
        [t1[k * _EB4:(k + 1) * _EB4] for k in range(4)], axis=1)


def _edge_mlp(edge_bf, edge_h, W1, b1, W2, b2):
    grid = _E_PAD // _EBLK
    return pl.pallas_call(
        _edge_mlp_body,
        grid=(grid,),
        in_specs=[
            pl.BlockSpec((_EBLK, _RAD), lambda i: (i, 0)),
            pl.BlockSpec((_IN, _EBLK), lambda i: (0, i)),
            pl.BlockSpec((_RAD, _IN), lambda i: (0, 0)),
            pl.BlockSpec((1, _IN), lambda i: (0, 0)),
            pl.BlockSpec((_IN, _IN), lambda i: (0, 0)),
            pl.BlockSpec((1, _IN), lambda i: (0, 0)),
        ],
        out_specs=[
            pl.BlockSpec((_EB4, 128), lambda i: (i, 0)),
            pl.BlockSpec((_EB4, 128), lambda i: (i, 0)),
        ],
        out_shape=[
            jax.ShapeDtypeStruct((_EP4, 128), jnp.float32),
            jax.ShapeDtypeStruct((_EP4, 128), jnp.float32),
        ],
    )(edge_bf, edge_h.T, W1, b1.reshape(1, _IN), W2, b2.reshape(1, _IN))


def _sc_body(xs0, xs1, eh0, eh1, src2, dst2, z32, z1, o1,
             s0_out, s1_out, c0_out, c1_out,
             sidx, didx, rows, ehv, ones_v, s_sh, cnt_sh,
             idx_sem, in_sem0, in_sem1, out_sem0, out_sem1):
    cid = lax.axis_index("c")
    sid = lax.axis_index("s")

    wr = _N_PAD // 16
    pltpu.sync_copy(z32.at[pl.ds(sid * wr, wr)], s_sh.at[pl.ds(sid * wr, wr)])
    pltpu.sync_copy(z1.at[pl.ds(sid * wr, wr)], cnt_sh.at[pl.ds(sid * wr, wr)])
    pltpu.sync_copy(o1, ones_v)
    plsc.subcore_barrier()

    row0 = sid * _ROWS_PER_SUB
    in_sems = (in_sem0, in_sem1)
    out_sems = (out_sem0, out_sem1)

    def fetch_idx(blk, b):
        rb = row0 + blk * _NCH
        pltpu.async_copy(src2.at[pl.ds(rb, _NCH)], sidx.at[b], idx_sem)
        pltpu.async_copy(dst2.at[pl.ds(rb, _NCH)], didx.at[b], idx_sem)

    def wait_idx(b):
        pltpu.make_async_copy(src2.at[pl.ds(0, _NCH)], sidx.at[b],
                              idx_sem).wait()
        pltpu.make_async_copy(dst2.at[pl.ds(0, _NCH)], didx.at[b],
                              idx_sem).wait()

    def do_half(x_tab, eh_tab, parity):

        def process(blk, ib):
            def issue_in(i, bb):
                g = pltpu.async_copy(x_tab.at[sidx.at[ib, i]], rows.at[bb],
                                     in_sems[bb])
                e = pltpu.async_copy(
                    eh_tab.at[pl.ds((row0 + blk * _NCH + i) * 32, 32)],
                    ehv.at[bb], in_sems[bb])
                return (g, e)

            def compute(bb):
                def mb(r, c):
                    for k in range(4):
                        e = r * 4 + k
                        rows[bb, e, pl.ds(0, 16)] = (
                            rows[bb, e, pl.ds(0, 16)]
                            * ehv[bb, r, pl.ds(k * 32, 16)])
                        rows[bb, e, pl.ds(16, 16)] = (
                            rows[bb, e, pl.ds(16, 16)]
                            * ehv[bb, r, pl.ds(k * 32 + 16, 16)])
                    return c
                lax.fori_loop(0, 32, mb, 0)

            descs = {}
            sdesc = {}
            descs[0] = issue_in(0, 0)
            for i in range(_NCH):
                b = i % 2
                nb = 1 - b
                if i < _NCH - 1:
                    if i >= 1:
                        sdesc[nb].wait()
                    descs[nb] = issue_in(i + 1, nb)
                for d in descs[b]:
                    d.wait()
                compute(b)
                sdesc[b] = pltpu.async_copy(rows.at[b],
                                            s_sh.at[didx.at[ib, i]],
                                            out_sems[b], add=True)
                if i % 2 == parity:
                    pltpu.sync_copy(ones_v, cnt_sh.at[didx.at[ib, i]],
                                    add=True)
            sdesc[0].wait()
            sdesc[1].wait()

        def blk2_body(t, c):
            wait_idx(0)
            fetch_idx(2 * t + 1, 1)
            process(2 * t, 0)
            wait_idx(1)

            @pl.when(t < _NBLK2 - 1)
            def _():
                fetch_idx(2 * t + 2, 0)

            process(2 * t + 1, 1)
            return c

        fetch_idx(0, 0)
        lax.fori_loop(0, _NBLK2, blk2_body, 0)

    @pl.when(cid == 0)
    def _():
        do_half(xs0, eh0, 0)

    @pl.when(cid == 1)
    def _():
        do_half(xs1, eh1, 1)

    plsc.subcore_barrier()

    @pl.when(cid == 0)
    def _():
        pltpu.sync_copy(s_sh.at[pl.ds(sid * wr, wr)],
                        s0_out.at[pl.ds(sid * wr, wr)])
        pltpu.sync_copy(cnt_sh.at[pl.ds(sid * wr, wr)],
                        c0_out.at[pl.ds(sid * wr, wr)])

    @pl.when(cid == 1)
    def _():
        pltpu.sync_copy(s_sh.at[pl.ds(sid * wr, wr)],
                        s1_out.at[pl.ds(sid * wr, wr)])
        pltpu.sync_copy(cnt_sh.at[pl.ds(sid * wr, wr)],
                        c1_out.at[pl.ds(sid * wr, wr)])


def _sc_gather_scatter(xs0, xs1, eh0, eh1, src2, dst2, z32, z1, o1):
    f32 = jnp.float32
    mesh = plsc.VectorSubcoreMesh(core_axis_name="c", subcore_axis_name="s")
    kern = pl.kernel(
        _sc_body,
        compiler_params=pltpu.CompilerParams(use_tc_tiling_on_sc=False),
        out_type=[
            jax.ShapeDtypeStruct((_N_PAD, _HALF), f32),
            jax.ShapeDtypeStruct((_N_PAD, _HALF), f32),
            jax.ShapeDtypeStruct((_N_PAD,), f32),
            jax.ShapeDtypeStruct((_N_PAD,), f32),
        ],
        mesh=mesh,
        scratch_types=[
            pltpu.VMEM((2, _NCH, 128), jnp.int32),
            pltpu.VMEM((2, _NCH, 128), jnp.int32),
            pltpu.VMEM((2, 128, _HALF), jnp.float32),
            pltpu.VMEM((2, 32, 128), jnp.float32),
            pltpu.VMEM((128,), jnp.float32),
            pltpu.VMEM_SHARED((_N_PAD, _HALF), f32),
            pltpu.VMEM_SHARED((_N_PAD,), f32),
            pltpu.SemaphoreType.DMA,
            pltpu.SemaphoreType.DMA,
            pltpu.SemaphoreType.DMA,
            pltpu.SemaphoreType.DMA,
            pltpu.SemaphoreType.DMA,
        ],
    )
    return kern(xs0, xs1, eh0, eh1, src2, dst2, z32, z1, o1)


def _out_mlp_body(s0_ref, s1_ref, c0_ref, c1_ref, w3_ref, b3_ref,
                  w4_ref, b4_ref, o_ref):
    cnt = c0_ref[...] + c1_ref[...]
    scale = 1.0 / jnp.maximum(cnt, 1.0)
    h = jnp.concatenate([s0_ref[...], s1_ref[...]], axis=1) * scale
    t = jnp.dot(h, w3_ref[...], preferred_element_type=jnp.float32)
    t = _ssp(t + b3_ref[...])
    t = jnp.dot(t, w4_ref[...], preferred_element_type=jnp.float32)
    o_ref[...] = _ssp(t + b4_ref[...])


def _out_mlp(s0, s1, c0, c1, W3, b3, W4, b4):
    grid = _N_NODES // _NBLK
    return pl.pallas_call(
        _out_mlp_body,
        grid=(grid,),
        in_specs=[
            pl.BlockSpec((_NBLK, _HALF), lambda i: (i, 0)),
            pl.BlockSpec((_NBLK, _HALF), lambda i: (i, 0)),
            pl.BlockSpec((_NBLK, 1), lambda i: (i, 0)),
            pl.BlockSpec((_NBLK, 1), lambda i: (i, 0)),
            pl.BlockSpec((_IN, _IN), lambda i: (0, 0)),
            pl.BlockSpec((1, _IN), lambda i: (0, 0)),
            pl.BlockSpec((_IN, _IN), lambda i: (0, 0)),
            pl.BlockSpec((1, _IN), lambda i: (0, 0)),
        ],
        out_specs=pl.BlockSpec((_NBLK, _IN), lambda i: (i, 0)),
        out_shape=jax.ShapeDtypeStruct((_N_NODES, _IN), jnp.float32),
    )(s0, s1, c0, c1, W3, b3.reshape(1, _IN), W4, b4.reshape(1, _IN))


def kernel(x, edge_bf, edge_h, edge_index, W1, b1, W2, b2, W3, b3, W4, b4):
    src = edge_index[0]
    dst = edge_index[1]
    pad = _E_PAD - _N_EDGES
    def _perm(a):
        a = a.reshape(_E_PAD // _EBLK, 4, _EB4 // 32, 32)
        return a.transpose(0, 2, 3, 1).reshape(_IDX_ROWS, 128)

    src_p = _perm(jnp.concatenate([src, jnp.zeros((pad,), jnp.int32)]))
    dst_p = _perm(jnp.concatenate([dst, jnp.full((pad,), _DUMP, jnp.int32)]))
    xs0 = x[:, :_HALF]
    xs1 = x[:, _HALF:]

    eh0, eh1 = _edge_mlp(edge_bf, edge_h, W1, b1, W2, b2)

    z32 = jnp.zeros((_N_PAD, _HALF), jnp.float32)
    z1 = jnp.zeros((_N_PAD,), jnp.float32)
    o1 = jnp.ones((128,), jnp.float32)
    s0, s1, c0, c1 = _sc_gather_scatter(
        xs0, xs1, eh0, eh1, src_p, dst_p, z32, z1, o1)
    c0 = c0.reshape(_N_PAD, 1)
    c1 = c1.reshape(_N_PAD, 1)

    return _out_mlp(s0, s1, c0, c1, W3, b3, W4, b4)


# speedup vs baseline: 4.6244x; 1.0384x over previous
import numpy as np
import jax
import jax.numpy as jnp
from jax import lax
from jax.experimental import pallas as pl
from jax.experimental.pallas import tpu as pltpu
from jax.experimental.pallas import tpu_sc as plsc

_N_NODES = 50000
_N_EDGES = 800000
_IN = 64
_RAD = 128
_HALF = 32
_LOG2 = float(np.log(2.0))

_E_PAD = 802816
_IDX_ROWS = _E_PAD // 128
_EP4 = _E_PAD // 4
_ROWS_PER_SUB = _IDX_ROWS // 16
_NCH = 14
_NBLK2 = _ROWS_PER_SUB // _NCH // 2
_N_PAD = 50048
_DUMP = _N_NODES

_EBLK = 7168
_EB4 = _EBLK // 4
_NBLK = 5000


def _ssp(v):
    return jnp.maximum(v, 0.0) + jnp.log(1.0 + jnp.exp(-jnp.abs(v))) - _LOG2


def _edge_mlp_body(bf_ref, eht_ref, w1_ref, b1_ref, w2_ref, b2_ref,
                   o0_ref, o1_ref):
    t = jnp.dot(bf_ref[...], w1_ref[...], preferred_element_type=jnp.float32)
    t = _ssp(t + b1_ref[...])
    t = jnp.dot(t, w2_ref[...], preferred_element_type=jnp.float32)
    t = _ssp(t + b2_ref[...]) * eht_ref[...].T
    t0 = t[:, :_HALF]
    t1 = t[:, _HALF:]
    o0_ref[...] = jnp.concatenate(
        [t0[k * _EB4:(k + 1) * _EB4] for k in range(4)], axis=1)
    o1_ref[...] = jnp.concatenate(
        [t1[k * _EB4:(k + 1) * _EB4] for k in range(4)], axis=1)


def _edge_mlp(edge_bf, edge_h, W1, b1, W2, b2):
    grid = _E_PAD // _EBLK
    return pl.pallas_call(
        _edge_mlp_body,
        grid=(grid,),
        in_specs=[
            pl.BlockSpec((_EBLK, _RAD), lambda i: (i, 0)),
            pl.BlockSpec((_IN, _EBLK), lambda i: (0, i)),
            pl.BlockSpec((_RAD, _IN), lambda i: (0, 0)),
            pl.BlockSpec((1, _IN), lambda i: (0, 0)),
            pl.BlockSpec((_IN, _IN), lambda i: (0, 0)),
            pl.BlockSpec((1, _IN), lambda i: (0, 0)),
        ],
        out_specs=[
            pl.BlockSpec((_EB4, 128), lambda i: (i, 0)),
            pl.BlockSpec((_EB4, 128), lambda i: (i, 0)),
        ],
        out_shape=[
            jax.ShapeDtypeStruct((_EP4, 128), jnp.float32),
            jax.ShapeDtypeStruct((_EP4, 128), jnp.float32),
        ],
    )(edge_bf, edge_h.T, W1, b1.reshape(1, _IN), W2, b2.reshape(1, _IN))


def _sc_body(xs0, xs1, eh0, eh1, src2, dst2, z32, z1, o1,
             s0_out, s1_out, c0_out, c1_out,
             sidx, didx, rows, ehv, ones_v, s_sh, cnt_sh,
             idx_sem, in_sem0, in_sem1, out_sem0, out_sem1, cnt_sem):
    cid = lax.axis_index("c")
    sid = lax.axis_index("s")

    wr = _N_PAD // 16
    pltpu.sync_copy(z32.at[pl.ds(sid * wr, wr)], s_sh.at[pl.ds(sid * wr, wr)])
    pltpu.sync_copy(z1.at[pl.ds(sid * wr, wr)], cnt_sh.at[pl.ds(sid * wr, wr)])
    pltpu.sync_copy(o1, ones_v)
    plsc.subcore_barrier()

    row0 = sid * _ROWS_PER_SUB
    in_sems = (in_sem0, in_sem1)
    out_sems = (out_sem0, out_sem1)

    def fetch_idx(blk, b):
        rb = row0 + blk * _NCH
        pltpu.async_copy(src2.at[pl.ds(rb, _NCH)], sidx.at[b], idx_sem)
        pltpu.async_copy(dst2.at[pl.ds(rb, _NCH)], didx.at[b], idx_sem)

    def wait_idx(b):
        pltpu.make_async_copy(src2.at[pl.ds(0, _NCH)], sidx.at[b],
                              idx_sem).wait()
        pltpu.make_async_copy(dst2.at[pl.ds(0, _NCH)], didx.at[b],
                              idx_sem).wait()

    def do_half(x_tab, eh_tab, parity):

        def process(blk, ib):
            def issue_in(i, bb):
                g = pltpu.async_copy(x_tab.at[sidx.at[ib, i]], rows.at[bb],
                                     in_sems[bb])
                e = pltpu.async_copy(
                    eh_tab.at[pl.ds((row0 + blk * _NCH + i) * 32, 32)],
                    ehv.at[bb], in_sems[bb])
                return (g, e)

            def compute(bb):
                def mb(r, c):
                    for k in range(4):
                        e = r * 4 + k
                        rows[bb, e, pl.ds(0, 16)] = (
                            rows[bb, e, pl.ds(0, 16)]
                            * ehv[bb, r, pl.ds(k * 32, 16)])
                        rows[bb, e, pl.ds(16, 16)] = (
                            rows[bb, e, pl.ds(16, 16)]
                            * ehv[bb, r, pl.ds(k * 32 + 16, 16)])
                    return c
                lax.fori_loop(0, 32, mb, 0)

            descs = {}
            sdesc = {}
            cdescs = []
            descs[0] = issue_in(0, 0)
            for i in range(_NCH):
                b = i % 2
                nb = 1 - b
                if i < _NCH - 1:
                    if i >= 1:
                        sdesc[nb].wait()
                    descs[nb] = issue_in(i + 1, nb)
                for d in descs[b]:
                    d.wait()
                compute(b)
                sdesc[b] = pltpu.async_copy(rows.at[b],
                                            s_sh.at[didx.at[ib, i]],
                                            out_sems[b], add=True)
                if i % 2 == parity:
                    cdescs.append(
                        pltpu.async_copy(ones_v, cnt_sh.at[didx.at[ib, i]],
                                         cnt_sem, add=True))
            sdesc[0].wait()
            sdesc[1].wait()
            for d in cdescs:
                d.wait()

        def blk2_body(t, c):
            wait_idx(0)
            fetch_idx(2 * t + 1, 1)
            process(2 * t, 0)
            wait_idx(1)

            @pl.when(t < _NBLK2 - 1)
            def _():
                fetch_idx(2 * t + 2, 0)

            process(2 * t + 1, 1)
            return c

        fetch_idx(0, 0)
        lax.fori_loop(0, _NBLK2, blk2_body, 0)

    @pl.when(cid == 0)
    def _():
        do_half(xs0, eh0, 0)

    @pl.when(cid == 1)
    def _():
        do_half(xs1, eh1, 1)

    plsc.subcore_barrier()

    @pl.when(cid == 0)
    def _():
        pltpu.sync_copy(s_sh.at[pl.ds(sid * wr, wr)],
                        s0_out.at[pl.ds(sid * wr, wr)])
        pltpu.sync_copy(cnt_sh.at[pl.ds(sid * wr, wr)],
                        c0_out.at[pl.ds(sid * wr, wr)])

    @pl.when(cid == 1)
    def _():
        pltpu.sync_copy(s_sh.at[pl.ds(sid * wr, wr)],
                        s1_out.at[pl.ds(sid * wr, wr)])
        pltpu.sync_copy(cnt_sh.at[pl.ds(sid * wr, wr)],
                        c1_out.at[pl.ds(sid * wr, wr)])


def _sc_gather_scatter(xs0, xs1, eh0, eh1, src2, dst2, z32, z1, o1):
    f32 = jnp.float32
    mesh = plsc.VectorSubcoreMesh(core_axis_name="c", subcore_axis_name="s")
    kern = pl.kernel(
        _sc_body,
        compiler_params=pltpu.CompilerParams(use_tc_tiling_on_sc=False),
        out_type=[
            jax.ShapeDtypeStruct((_N_PAD, _HALF), f32),
            jax.ShapeDtypeStruct((_N_PAD, _HALF), f32),
            jax.ShapeDtypeStruct((_N_PAD,), f32),
            jax.ShapeDtypeStruct((_N_PAD,), f32),
        ],
        mesh=mesh,
        scratch_types=[
            pltpu.VMEM((2, _NCH, 128), jnp.int32),
            pltpu.VMEM((2, _NCH, 128), jnp.int32),
            pltpu.VMEM((2, 128, _HALF), jnp.float32),
            pltpu.VMEM((2, 32, 128), jnp.float32),
            pltpu.VMEM((128,), jnp.float32),
            pltpu.VMEM_SHARED((_N_PAD, _HALF), f32),
            pltpu.VMEM_SHARED((_N_PAD,), f32),
            pltpu.SemaphoreType.DMA,
            pltpu.SemaphoreType.DMA,
            pltpu.SemaphoreType.DMA,
            pltpu.SemaphoreType.DMA,
            pltpu.SemaphoreType.DMA,
            pltpu.SemaphoreType.DMA,
        ],
    )
    return kern(xs0, xs1, eh0, eh1, src2, dst2, z32, z1, o1)


def _out_mlp_body(s0_ref, s1_ref, c0_ref, c1_ref, w3_ref, b3_ref,
                  w4_ref, b4_ref, o_ref):
    cnt = c0_ref[...] + c1_ref[...]
    scale = 1.0 / jnp.maximum(cnt, 1.0)
    h = jnp.concatenate([s0_ref[...], s1_ref[...]], axis=1) * scale
    t = jnp.dot(h, w3_ref[...], preferred_element_type=jnp.float32)
    t = _ssp(t + b3_ref[...])
    t = jnp.dot(t, w4_ref[...], preferred_element_type=jnp.float32)
    o_ref[...] = _ssp(t + b4_ref[...])


def _out_mlp(s0, s1, c0, c1, W3, b3, W4, b4):
    grid = _N_NODES // _NBLK
    return pl.pallas_call(
        _out_mlp_body,
        grid=(grid,),
        in_specs=[
            pl.BlockSpec((_NBLK, _HALF), lambda i: (i, 0)),
            pl.BlockSpec((_NBLK, _HALF), lambda i: (i, 0)),
            pl.BlockSpec((_NBLK, 1), lambda i: (i, 0)),
            pl.BlockSpec((_NBLK, 1), lambda i: (i, 0)),
            pl.BlockSpec((_IN, _IN), lambda i: (0, 0)),
            pl.BlockSpec((1, _IN), lambda i: (0, 0)),
            pl.BlockSpec((_IN, _IN), lambda i: (0, 0)),
            pl.BlockSpec((1, _IN), lambda i: (0, 0)),
        ],
        out_specs=pl.BlockSpec((_NBLK, _IN), lambda i: (i, 0)),
        out_shape=jax.ShapeDtypeStruct((_N_NODES, _IN), jnp.float32),
    )(s0, s1, c0, c1, W3, b3.reshape(1, _IN), W4, b4.reshape(1, _IN))


def kernel(x, edge_bf, edge_h, edge_index, W1, b1, W2, b2, W3, b3, W4, b4):
    src = edge_index[0]
    dst = edge_index[1]
    pad = _E_PAD - _N_EDGES
    def _perm(a):
        a = a.reshape(_E_PAD // _EBLK, 4, _EB4 // 32, 32)
        return a.transpose(0, 2, 3, 1).reshape(_IDX_ROWS, 128)

    src_p = _perm(jnp.concatenate([src, jnp.zeros((pad,), jnp.int32)]))
    dst_p = _perm(jnp.concatenate([dst, jnp.full((pad,), _DUMP, jnp.int32)]))
    xs0 = x[:, :_HALF]
    xs1 = x[:, _HALF:]

    eh0, eh1 = _edge_mlp(edge_bf, edge_h, W1, b1, W2, b2)

    z32 = jnp.zeros((_N_PAD, _HALF), jnp.float32)
    z1 = jnp.zeros((_N_PAD,), jnp.float32)
    o1 = jnp.ones((128,), jnp.float32)
    s0, s1, c0, c1 = _sc_gather_scatter(
        xs0, xs1, eh0, eh1, src_p, dst_p, z32, z1, o1)
    c0 = c0.reshape(_N_PAD, 1)
    c1 = c1.reshape(_N_PAD, 1)

    return _out_mlp(s0, s1, c0, c1, W3, b3, W4, b4)


# speedup vs baseline: 5.4377x; 1.1759x over previous
import numpy as np
import jax
import jax.numpy as jnp
from jax import lax
from jax.experimental import pallas as pl
from jax.experimental.pallas import tpu as pltpu
from jax.experimental.pallas import tpu_sc as plsc

_N_NODES = 50000
_N_EDGES = 800000
_IN = 64
_RAD = 128
_HALF = 32
_LOG2 = float(np.log(2.0))

_E_PAD = 802816
_IDX_ROWS = _E_PAD // 128
_E_HALF = _E_PAD // 2
_IDX_ROWS_H = _IDX_ROWS // 2
_EP4H = _E_HALF // 4
_ROWS_PER_SUB = _IDX_ROWS_H // 16
_NCH = 14
_NBLK2 = _ROWS_PER_SUB // _NCH // 2
_N_PAD = 50048
_DUMP = _N_NODES

_EBLK = 7168
_EB4 = _EBLK // 4
_NBLK = 5000


def _ssp(v):
    return jnp.maximum(v, 0.0) + jnp.log(1.0 + jnp.exp(-jnp.abs(v))) - _LOG2


def _edge_mlp_body(bf_ref, eht_ref, w1_ref, b1_ref, w2_ref, b2_ref,
                   o0_ref, o1_ref):
    t = jnp.dot(bf_ref[...], w1_ref[...], preferred_element_type=jnp.float32)
    t = _ssp(t + b1_ref[...])
    t = jnp.dot(t, w2_ref[...], preferred_element_type=jnp.float32)
    t = _ssp(t + b2_ref[...]) * eht_ref[...].T
    t0 = t[:, :_HALF]
    t1 = t[:, _HALF:]
    o0_ref[...] = jnp.concatenate(
        [t0[k * _EB4:(k + 1) * _EB4] for k in range(4)], axis=1)
    o1_ref[...] = jnp.concatenate(
        [t1[k * _EB4:(k + 1) * _EB4] for k in range(4)], axis=1)


def _edge_mlp(edge_bf, edge_ht, W1, b1, W2, b2, off):
    grid = _E_HALF // _EBLK
    return pl.pallas_call(
        _edge_mlp_body,
        grid=(grid,),
        in_specs=[
            pl.BlockSpec((_EBLK, _RAD), lambda i: (i + off, 0)),
            pl.BlockSpec((_IN, _EBLK), lambda i: (0, i + off)),
            pl.BlockSpec((_RAD, _IN), lambda i: (0, 0)),
            pl.BlockSpec((1, _IN), lambda i: (0, 0)),
            pl.BlockSpec((_IN, _IN), lambda i: (0, 0)),
            pl.BlockSpec((1, _IN), lambda i: (0, 0)),
        ],
        out_specs=[
            pl.BlockSpec((_EB4, 128), lambda i: (i, 0)),
            pl.BlockSpec((_EB4, 128), lambda i: (i, 0)),
        ],
        out_shape=[
            jax.ShapeDtypeStruct((_EP4H, 128), jnp.float32),
            jax.ShapeDtypeStruct((_EP4H, 128), jnp.float32),
        ],
    )(edge_bf, edge_ht, W1, b1.reshape(1, _IN), W2, b2.reshape(1, _IN))


def _sc_body(xs, eh0, eh1, src2, dst2, s0i, s1i, c0i, c1i, o1,
             s0_out, s1_out, c0_out, c1_out,
             sidx, didx, rows, ehv, ones_v, s_sh, cnt_sh,
             idx_sem, in_sem0, in_sem1, out_sem0, out_sem1, cnt_sem):
    cid = lax.axis_index("c")
    sid = lax.axis_index("s")

    wr = _N_PAD // 16

    @pl.when(cid == 0)
    def _():
        pltpu.sync_copy(s0i.at[pl.ds(sid * wr, wr)],
                        s_sh.at[pl.ds(sid * wr, wr)])
        pltpu.sync_copy(c0i.at[pl.ds(sid * wr, wr)],
                        cnt_sh.at[pl.ds(sid * wr, wr)])

    @pl.when(cid == 1)
    def _():
        pltpu.sync_copy(s1i.at[pl.ds(sid * wr, wr)],
                        s_sh.at[pl.ds(sid * wr, wr)])
        pltpu.sync_copy(c1i.at[pl.ds(sid * wr, wr)],
                        cnt_sh.at[pl.ds(sid * wr, wr)])

    pltpu.sync_copy(o1, ones_v)
    plsc.subcore_barrier()

    row0 = sid * _ROWS_PER_SUB
    in_sems = (in_sem0, in_sem1)
    out_sems = (out_sem0, out_sem1)

    def fetch_idx(blk, b):
        rb = row0 + blk * _NCH
        pltpu.async_copy(src2.at[pl.ds(rb, _NCH)], sidx.at[b], idx_sem)
        pltpu.async_copy(dst2.at[pl.ds(rb, _NCH)], didx.at[b], idx_sem)

    def wait_idx(b):
        pltpu.make_async_copy(src2.at[pl.ds(0, _NCH)], sidx.at[b],
                              idx_sem).wait()
        pltpu.make_async_copy(dst2.at[pl.ds(0, _NCH)], didx.at[b],
                              idx_sem).wait()

        def remap(i, c):
            for k in range(8):
                sidx[b, i, pl.ds(k * 16, 16)] = (
                    sidx[b, i, pl.ds(k * 16, 16)] * 2 + cid)
            return c
        lax.fori_loop(0, _NCH, remap, 0)

    def do_half(eh_tab, parity):

        def process(blk, ib):
            def issue_in(i, bb):
                g = pltpu.async_copy(xs.at[sidx.at[ib, i]], rows.at[bb],
                                     in_sems[bb])
                e = pltpu.async_copy(
                    eh_tab.at[pl.ds((row0 + blk * _NCH + i) * 32, 32)],
                    ehv.at[bb], in_sems[bb])
                return (g, e)

            def compute(bb):
                def mb(r, c):
                    for k in range(4):
                        e = r * 4 + k
                        rows[bb, e, pl.ds(0, 16)] = (
                            rows[bb, e, pl.ds(0, 16)]
                            * ehv[bb, r, pl.ds(k * 32, 16)])
                        rows[bb, e, pl.ds(16, 16)] = (
                            rows[bb, e, pl.ds(16, 16)]
                            * ehv[bb, r, pl.ds(k * 32 + 16, 16)])
                    return c
                lax.fori_loop(0, 32, mb, 0)

            descs = {}
            sdesc = {}
            cdescs = []
            descs[0] = issue_in(0, 0)
            for i in range(_NCH):
                b = i % 2
                nb = 1 - b
                if i < _NCH - 1:
                    if i >= 1:
                        sdesc[nb].wait()
                    descs[nb] = issue_in(i + 1, nb)
                for d in descs[b]:
                    d.wait()
                compute(b)
                sdesc[b] = pltpu.async_copy(rows.at[b],
                                            s_sh.at[didx.at[ib, i]],
                                            out_sems[b], add=True)
                if i % 2 == parity:
                    cdescs.append(
                        pltpu.async_copy(ones_v, cnt_sh.at[didx.at[ib, i]],
                                         cnt_sem, add=True))
            sdesc[0].wait()
            sdesc[1].wait()
            for d in cdescs:
                d.wait()

        def blk2_body(t, c):
            wait_idx(0)
            fetch_idx(2 * t + 1, 1)
            process(2 * t, 0)
            wait_idx(1)

            @pl.when(t < _NBLK2 - 1)
            def _():
                fetch_idx(2 * t + 2, 0)

            process(2 * t + 1, 1)
            return c

        fetch_idx(0, 0)
        lax.fori_loop(0, _NBLK2, blk2_body, 0)

    @pl.when(cid == 0)
    def _():
        do_half(eh0, 0)

    @pl.when(cid == 1)
    def _():
        do_half(eh1, 1)

    plsc.subcore_barrier()

    @pl.when(cid == 0)
    def _():
        pltpu.sync_copy(s_sh.at[pl.ds(sid * wr, wr)],
                        s0_out.at[pl.ds(sid * wr, wr)])
        pltpu.sync_copy(cnt_sh.at[pl.ds(sid * wr, wr)],
                        c0_out.at[pl.ds(sid * wr, wr)])

    @pl.when(cid == 1)
    def _():
        pltpu.sync_copy(s_sh.at[pl.ds(sid * wr, wr)],
                        s1_out.at[pl.ds(sid * wr, wr)])
        pltpu.sync_copy(cnt_sh.at[pl.ds(sid * wr, wr)],
                        c1_out.at[pl.ds(sid * wr, wr)])


def _sc_gather_scatter(xs, eh0, eh1, src2, dst2, s0i, s1i, c0i, c1i, o1):
    f32 = jnp.float32
    mesh = plsc.VectorSubcoreMesh(core_axis_name="c", subcore_axis_name="s")
    kern = pl.kernel(
        _sc_body,
        compiler_params=pltpu.CompilerParams(use_tc_tiling_on_sc=False),
        out_type=[
            jax.ShapeDtypeStruct((_N_PAD, _HALF), f32),
            jax.ShapeDtypeStruct((_N_PAD, _HALF), f32),
            jax.ShapeDtypeStruct((_N_PAD,), f32),
            jax.ShapeDtypeStruct((_N_PAD,), f32),
        ],
        mesh=mesh,
        scratch_types=[
            pltpu.VMEM((2, _NCH, 128), jnp.int32),
            pltpu.VMEM((2, _NCH, 128), jnp.int32),
            pltpu.VMEM((2, 128, _HALF), jnp.float32),
            pltpu.VMEM((2, 32, 128), jnp.float32),
            pltpu.VMEM((128,), jnp.float32),
            pltpu.VMEM_SHARED((_N_PAD, _HALF), f32),
            pltpu.VMEM_SHARED((_N_PAD,), f32),
            pltpu.SemaphoreType.DMA,
            pltpu.SemaphoreType.DMA,
            pltpu.SemaphoreType.DMA,
            pltpu.SemaphoreType.DMA,
            pltpu.SemaphoreType.DMA,
            pltpu.SemaphoreType.DMA,
        ],
    )
    return kern(xs, eh0, eh1, src2, dst2, s0i, s1i, c0i, c1i, o1)


def _out_mlp_body(s0_ref, s1_ref, c0_ref, c1_ref, w3_ref, b3_ref,
                  w4_ref, b4_ref, o_ref):
    cnt = c0_ref[...] + c1_ref[...]
    scale = 1.0 / jnp.maximum(cnt, 1.0)
    h = jnp.concatenate([s0_ref[...], s1_ref[...]], axis=1) * scale
    t = jnp.dot(h, w3_ref[...], preferred_element_type=jnp.float32)
    t = _ssp(t + b3_ref[...])
    t = jnp.dot(t, w4_ref[...], preferred_element_type=jnp.float32)
    o_ref[...] = _ssp(t + b4_ref[...])


def _out_mlp(s0, s1, c0, c1, W3, b3, W4, b4):
    grid = _N_NODES // _NBLK
    return pl.pallas_call(
        _out_mlp_body,
        grid=(grid,),
        in_specs=[
            pl.BlockSpec((_NBLK, _HALF), lambda i: (i, 0)),
            pl.BlockSpec((_NBLK, _HALF), lambda i: (i, 0)),
            pl.BlockSpec((_NBLK, 1), lambda i: (i, 0)),
            pl.BlockSpec((_NBLK, 1), lambda i: (i, 0)),
            pl.BlockSpec((_IN, _IN), lambda i: (0, 0)),
            pl.BlockSpec((1, _IN), lambda i: (0, 0)),
            pl.BlockSpec((_IN, _IN), lambda i: (0, 0)),
            pl.BlockSpec((1, _IN), lambda i: (0, 0)),
        ],
        out_specs=pl.BlockSpec((_NBLK, _IN), lambda i: (i, 0)),
        out_shape=jax.ShapeDtypeStruct((_N_NODES, _IN), jnp.float32),
    )(s0, s1, c0, c1, W3, b3.reshape(1, _IN), W4, b4.reshape(1, _IN))


def kernel(x, edge_bf, edge_h, edge_index, W1, b1, W2, b2, W3, b3, W4, b4):
    src = edge_index[0]
    dst = edge_index[1]
    pad = _E_PAD - _N_EDGES
    def _perm(a):
        a = a.reshape(_E_PAD // _EBLK, 4, _EB4 // 32, 32)
        return a.transpose(0, 2, 3, 1).reshape(_IDX_ROWS, 128)

    src_p = _perm(jnp.concatenate([src, jnp.zeros((pad,), jnp.int32)]))
    dst_p = _perm(jnp.concatenate([dst, jnp.full((pad,), _DUMP, jnp.int32)]))
    xs = x.reshape(2 * _N_NODES, _HALF)

    edge_ht = edge_h.T
    ehA0, ehA1 = _edge_mlp(edge_bf, edge_ht, W1, b1, W2, b2, 0)
    ehB0, ehB1 = _edge_mlp(edge_bf, edge_ht, W1, b1, W2, b2,
                           _E_HALF // _EBLK)

    z32 = jnp.zeros((_N_PAD, _HALF), jnp.float32)
    z1 = jnp.zeros((_N_PAD,), jnp.float32)
    o1 = jnp.ones((128,), jnp.float32)
    sA0, sA1, cA0, cA1 = _sc_gather_scatter(
        xs, ehA0, ehA1, src_p[:_IDX_ROWS_H], dst_p[:_IDX_ROWS_H],
        z32, z32, z1, z1, o1)
    s0, s1, c0, c1 = _sc_gather_scatter(
        xs, ehB0, ehB1, src_p[_IDX_ROWS_H:], dst_p[_IDX_ROWS_H:],
        sA0, sA1, cA0, cA1, o1)
    c0 = c0.reshape(_N_PAD, 1)
    c1 = c1.reshape(_N_PAD, 1)

    return _out_mlp(s0, s1, c0, c1, W3, b3, W4, b4)
